# baseline (speedup 1.0000x reference)
import functools

import jax
import jax.numpy as jnp
from jax import lax
from jax.experimental import pallas as pl
from jax.experimental.pallas import tpu as pltpu
from jax.experimental.pallas import tpu_sc as plsc

BS = 8
H = W = 512
L = H * W
T = BS * L
NQ = 4
QSZ = L // NQ
CH = 128
NCHUNK = QSZ // CH
NV = CH // 16
ROWS = T // CH
ALPHA_C = 0.1
POS_ALL_NEG = 5.0
NEG_ALL_POS = -5.0


def _dilate_body(t_ref, o_ref):
    t2 = t_ref[0]
    b = (t2 > 0.0).astype(jnp.float32)
    zr = jnp.zeros((3, W), jnp.float32)
    xp = jnp.concatenate([zr, b, zr], axis=0)
    r = xp[0:H]
    for d in range(1, 7):
        r = jnp.maximum(r, xp[d:d + H])
    zc = jnp.zeros((H, 3), jnp.float32)
    xp2 = jnp.concatenate([zc, r, zc], axis=1)
    r2 = xp2[:, 0:W]
    for d in range(1, 7):
        r2 = jnp.maximum(r2, xp2[:, d:d + W])
    o_ref[0] = (r2 > 0.0).astype(jnp.float32)


def _dilate(t3):
    return pl.pallas_call(
        _dilate_body,
        grid=(BS,),
        in_specs=[pl.BlockSpec((1, H, W), lambda i: (i, 0, 0))],
        out_specs=pl.BlockSpec((1, H, W), lambda i: (i, 0, 0)),
        out_shape=jax.ShapeDtypeStruct((BS, H, W), jnp.float32),
    )(t3)


def _lane_extract(vec, k):
    return jnp.sum(jnp.where(lax.broadcasted_iota(jnp.int32, (16,), 0) == k,
                             vec, 0))


def _sc_ids():
    cid = lax.axis_index("c")
    sid = lax.axis_index("s")
    m = cid * 4 + sid // 4
    q = sid % 4
    return m, q


def _sc_p1_body(xf, tf, af, posbuf, fnbuf, counts,
                xb, tb, ab, ibp, ibf, ibn, cntv, sem1):
    m, q = _sc_ids()
    base_row = m * (L // CH) + q * (QSZ // CH)
    pos_base = m * L + q * QSZ
    neg_base = T + m * L + q * QSZ
    trash_p = pos_base + QSZ - 1
    trash_n = neg_base + QSZ - 1
    lane = lax.broadcasted_iota(jnp.int32, (16,), 0)

    def p1_body(c0, carry):
        cp, cf, cn = carry
        row = base_row + c0
        pltpu.sync_copy(xf.at[row], xb)
        pltpu.sync_copy(tf.at[row], tb)
        pltpu.sync_copy(af.at[row], ab)
        for v in range(NV):
            sl = pl.ds(16 * v, 16)
            xv = xb[sl]
            tv = tb[sl]
            av = ab[sl]
            mpos = tv > 0.0
            mneg = av <= 0.0
            mfp = jnp.logical_and(xv > 0.0, mneg)
            ip = jnp.where(mpos, jnp.int32(1), jnp.int32(0))
            if_ = jnp.where(mfp, jnp.int32(1), jnp.int32(0))
            in_ = jnp.where(mneg, jnp.int32(1), jnp.int32(0))
            cump = jnp.cumsum(ip)
            cumf = jnp.cumsum(if_)
            cumn = jnp.cumsum(in_)
            ibp[sl] = jnp.where(mpos, pos_base + cp + cump - 1, trash_p)
            ibf[sl] = jnp.where(mfp, pos_base + cf + cumf - 1, trash_p)
            ibn[sl] = jnp.where(mneg, neg_base + cn + cumn - 1, trash_n)
            cp = cp + jnp.sum(ip)
            cf = cf + jnp.sum(if_)
            cn = cn + jnp.sum(in_)
        pltpu.async_copy(xb, posbuf.at[ibp], sem1).wait()
        pltpu.async_copy(xb, fnbuf.at[ibf], sem1).wait()
        pltpu.async_copy(xb, fnbuf.at[ibn], sem1).wait()
        return cp, cf, cn

    z = jnp.int32(0)
    cp, cf, cn = lax.fori_loop(0, NCHUNK, p1_body, (z, z, z))
    cv = jnp.where(lane == 0, cp,
                   jnp.where(lane == 1, cf,
                             jnp.where(lane == 2, cn, 0)))
    cntv[...] = cv
    pltpu.sync_copy(cntv, counts.at[m * NQ + q])


def _sc_p2_body(posbuf, fnbuf, counts, dup_pos, dup_fp,
                ibp, ibf, gb1, gb2, rowb, sem1):
    m, q = _sc_ids()
    base_row = m * (L // CH) + q * (QSZ // CH)
    lane = lax.broadcasted_iota(jnp.int32, (16,), 0)

    for r in range(NQ):
        pltpu.sync_copy(counts.at[m * NQ + r], rowb.at[r])
    P = []
    F = []
    N = []
    for r in range(NQ):
        rv = rowb[r]
        P.append(_lane_extract(rv, 0))
        F.append(_lane_extract(rv, 1))
        N.append(_lane_extract(rv, 2))

    np_tot = P[0] + P[1] + P[2] + P[3]
    nfp_tot = F[0] + F[1] + F[2] + F[3]
    nneg_tot = N[0] + N[1] + N[2] + N[3]
    pc1, pc2, pc3 = P[0], P[0] + P[1], P[0] + P[1] + P[2]
    usefp = nfp_tot > 0
    c1 = jnp.where(usefp, F[0], N[0])
    c2 = c1 + jnp.where(usefp, F[1], N[1])
    c3 = c2 + jnp.where(usefp, F[2], N[2])
    ch_tot = jnp.where(usefp, nfp_tot, nneg_tot)
    ch_extra = jnp.where(usefp, 0, T)
    npc = jnp.maximum(np_tot, 1)
    chc = jnp.maximum(ch_tot, 1)

    def p2_body(c0, carry):
        for v in range(NV):
            sl = pl.ds(16 * v, 16)
            gi = q * QSZ + c0 * CH + 16 * v + lane
            jp = lax.rem(gi, npc)
            cbp = jnp.where(jp >= pc3, pc3,
                            jnp.where(jp >= pc2, pc2,
                                      jnp.where(jp >= pc1, pc1, 0)))
            one = jnp.int32(1)
            zero = jnp.int32(0)
            qp = (jnp.where(jp >= pc1, one, zero)
                  + jnp.where(jp >= pc2, one, zero)
                  + jnp.where(jp >= pc3, one, zero))
            ibp[sl] = m * L + qp * QSZ + (jp - cbp)
            jc = lax.rem(gi, chc)
            cbc = jnp.where(jc >= c3, c3,
                            jnp.where(jc >= c2, c2,
                                      jnp.where(jc >= c1, c1, 0)))
            qc = (jnp.where(jc >= c1, one, zero)
                  + jnp.where(jc >= c2, one, zero)
                  + jnp.where(jc >= c3, one, zero))
            ibf[sl] = ch_extra + m * L + qc * QSZ + (jc - cbc)
        pltpu.async_copy(posbuf.at[ibp], gb1, sem1).wait()
        pltpu.async_copy(fnbuf.at[ibf], gb2, sem1).wait()
        for v in range(NV):
            sl = pl.ds(16 * v, 16)
            g1 = gb1[sl]
            g2 = gb2[sl]
            gb1[sl] = jnp.where(np_tot > 0, g1, POS_ALL_NEG)
            gb2[sl] = jnp.where(nneg_tot > 0, g2, NEG_ALL_POS)
        row = base_row + c0
        pltpu.sync_copy(gb1, dup_pos.at[row])
        pltpu.sync_copy(gb2, dup_fp.at[row])
        return carry

    lax.fori_loop(0, NCHUNK, p2_body, jnp.int32(0))


def _sc_dup(xf2, tf2, af2):
    mesh = plsc.VectorSubcoreMesh(core_axis_name="c", subcore_axis_name="s")
    cparams = pltpu.CompilerParams(needs_layout_passes=False)
    p1 = pl.kernel(
        _sc_p1_body,
        out_type=[
            jax.ShapeDtypeStruct((T,), jnp.float32),
            jax.ShapeDtypeStruct((2 * T,), jnp.float32),
            jax.ShapeDtypeStruct((BS * NQ, 16), jnp.int32),
        ],
        mesh=mesh,
        scratch_types=[
            pltpu.VMEM((CH,), jnp.float32),
            pltpu.VMEM((CH,), jnp.float32),
            pltpu.VMEM((CH,), jnp.float32),
            pltpu.VMEM((CH,), jnp.int32),
            pltpu.VMEM((CH,), jnp.int32),
            pltpu.VMEM((CH,), jnp.int32),
            pltpu.VMEM((16,), jnp.int32),
            pltpu.SemaphoreType.DMA,
        ],
        compiler_params=cparams,
    )
    posbuf, fnbuf, counts = p1(xf2, tf2, af2)
    p2 = pl.kernel(
        _sc_p2_body,
        out_type=[
            jax.ShapeDtypeStruct((ROWS, CH), jnp.float32),
            jax.ShapeDtypeStruct((ROWS, CH), jnp.float32),
        ],
        mesh=mesh,
        scratch_types=[
            pltpu.VMEM((CH,), jnp.int32),
            pltpu.VMEM((CH,), jnp.int32),
            pltpu.VMEM((CH,), jnp.float32),
            pltpu.VMEM((CH,), jnp.float32),
            pltpu.VMEM((NQ, 16), jnp.int32),
            pltpu.SemaphoreType.DMA,
        ],
        compiler_params=cparams,
    )
    dup_pos, dup_fp = p2(posbuf, fnbuf, counts)
    return posbuf, fnbuf, dup_pos, dup_fp


def _bce_body(x_ref, t_ref, dp_ref, df_ref, o_ref):
    i = pl.program_id(0)
    x = x_ref[...]
    t = t_ref[...]
    dp = dp_ref[...]
    df = df_ref[...]
    sim = dp * x
    t1 = jnp.maximum(sim, 0.0) - sim * t + jnp.log1p(jnp.exp(-jnp.abs(sim)))
    t2 = jnp.maximum(dp, 0.0) - dp + jnp.log1p(jnp.exp(-jnp.abs(dp)))
    a = dp * df
    t3 = jnp.maximum(a, 0.0) + jnp.log1p(jnp.exp(-jnp.abs(a)))
    blk = x.shape[0]
    p1 = jnp.sum(t1.reshape(blk // 8, 8, 128), axis=0)
    p2 = jnp.sum(t2.reshape(blk // 8, 8, 128), axis=0)
    p3 = jnp.sum(t3.reshape(blk // 8, 8, 128), axis=0)
    acc = jnp.stack([p1, p2, p3])

    @pl.when(i == 0)
    def _():
        o_ref[...] = acc

    @pl.when(i > 0)
    def _():
        o_ref[...] = o_ref[...] + acc


def _bce_sums(x2, t2, dp2, df2):
    blk = 1024
    grid = ROWS // blk
    bs = pl.BlockSpec((blk, CH), lambda i: (i, 0))
    return pl.pallas_call(
        _bce_body,
        grid=(grid,),
        in_specs=[bs, bs, bs, bs],
        out_specs=pl.BlockSpec((3, 8, 128), lambda i: (0, 0, 0)),
        out_shape=jax.ShapeDtypeStruct((3, 8, 128), jnp.float32),
    )(x2, t2, dp2, df2)


def kernel(input, target):
    x = input.astype(jnp.float32)
    t = target.astype(jnp.float32)
    aug = _dilate(t.reshape(BS, H, W))
    x2 = x.reshape(ROWS, CH)
    t2 = t.reshape(ROWS, CH)
    a2 = aug.reshape(ROWS, CH)
    _, _, dp2, df2 = _sc_dup(x2, t2, a2)
    sums = _bce_sums(x2, t2, dp2, df2)
    s = jnp.sum(sums, axis=(1, 2))
    return (s[0] + s[1] + ALPHA_C * s[2]) / jnp.float32(T)


# speedup vs baseline: 1.0592x; 1.0592x over previous
import functools

import jax
import jax.numpy as jnp
from jax import lax
from jax.experimental import pallas as pl
from jax.experimental.pallas import tpu as pltpu
from jax.experimental.pallas import tpu_sc as plsc

BS = 8
H = W = 512
L = H * W
T = BS * L
NQ = 4
QSZ = L // NQ
CH = 128
NCHUNK = QSZ // CH
NV = CH // 16
CHB = 512
KB = CHB // CH
ROWS = T // CH
ALPHA_C = 0.1
POS_ALL_NEG = 5.0
NEG_ALL_POS = -5.0


def _dilate_body(t_ref, o_ref):
    t2 = t_ref[0]
    b = (t2 > 0.0).astype(jnp.float32)
    zr = jnp.zeros((3, W), jnp.float32)
    xp = jnp.concatenate([zr, b, zr], axis=0)
    r = xp[0:H]
    for d in range(1, 7):
        r = jnp.maximum(r, xp[d:d + H])
    zc = jnp.zeros((H, 3), jnp.float32)
    xp2 = jnp.concatenate([zc, r, zc], axis=1)
    r2 = xp2[:, 0:W]
    for d in range(1, 7):
        r2 = jnp.maximum(r2, xp2[:, d:d + W])
    o_ref[0] = (r2 > 0.0).astype(jnp.float32)


def _dilate(t3):
    return pl.pallas_call(
        _dilate_body,
        grid=(BS,),
        in_specs=[pl.BlockSpec((1, H, W), lambda i: (i, 0, 0))],
        out_specs=pl.BlockSpec((1, H, W), lambda i: (i, 0, 0)),
        out_shape=jax.ShapeDtypeStruct((BS, H, W), jnp.float32),
    )(t3)


def _lane_extract(vec, k):
    return jnp.sum(jnp.where(lax.broadcasted_iota(jnp.int32, (16,), 0) == k,
                             vec, 0))


def _sc_ids():
    cid = lax.axis_index("c")
    sid = lax.axis_index("s")
    m = cid * 4 + sid // 4
    q = sid % 4
    return m, q


def _sc_p1_body(xf, tf, af, posbuf, fnbuf, counts,
                xb, tb, ab, ibp, ibf, ibn, cntv, sem1):
    m, q = _sc_ids()
    base_row = m * (L // CH) + q * (QSZ // CH)
    pos_base = m * L + q * QSZ
    neg_base = T + m * L + q * QSZ
    trash_p = pos_base + QSZ - 1
    trash_n = neg_base + QSZ - 1
    lane = lax.broadcasted_iota(jnp.int32, (16,), 0)

    def p1_body(c0, carry):
        cp, cf, cn = carry
        row = base_row + c0 * KB
        h1 = pltpu.async_copy(xf.at[pl.ds(row, KB)], xb, sem1)
        h2 = pltpu.async_copy(tf.at[pl.ds(row, KB)], tb, sem1)
        h3 = pltpu.async_copy(af.at[pl.ds(row, KB)], ab, sem1)
        h1.wait()
        h2.wait()
        h3.wait()
        for k in range(KB):
            for v in range(NV):
                sl = pl.ds(16 * v, 16)
                xv = xb[k, sl]
                tv = tb[k, sl]
                av = ab[k, sl]
                mpos = tv > 0.0
                mneg = av <= 0.0
                mfp = jnp.logical_and(xv > 0.0, mneg)
                ip = jnp.where(mpos, jnp.int32(1), jnp.int32(0))
                if_ = jnp.where(mfp, jnp.int32(1), jnp.int32(0))
                in_ = jnp.where(mneg, jnp.int32(1), jnp.int32(0))
                cump = jnp.cumsum(ip)
                cumf = jnp.cumsum(if_)
                cumn = jnp.cumsum(in_)
                ibp[k, sl] = jnp.where(mpos, pos_base + cp + cump - 1, trash_p)
                ibf[k, sl] = jnp.where(mfp, pos_base + cf + cumf - 1, trash_p)
                ibn[k, sl] = jnp.where(mneg, neg_base + cn + cumn - 1, trash_n)
                cp = cp + jnp.sum(ip)
                cf = cf + jnp.sum(if_)
                cn = cn + jnp.sum(in_)
        hs = []
        for k in range(KB):
            hs.append(pltpu.async_copy(xb.at[k], posbuf.at[ibp.at[k]], sem1))
            hs.append(pltpu.async_copy(xb.at[k], fnbuf.at[ibf.at[k]], sem1))
            hs.append(pltpu.async_copy(xb.at[k], fnbuf.at[ibn.at[k]], sem1))
        for h in hs:
            h.wait()
        return cp, cf, cn

    z = jnp.int32(0)
    cp, cf, cn = lax.fori_loop(0, QSZ // CHB, p1_body, (z, z, z))
    cv = jnp.where(lane == 0, cp,
                   jnp.where(lane == 1, cf,
                             jnp.where(lane == 2, cn, 0)))
    cntv[...] = cv
    pltpu.sync_copy(cntv, counts.at[m * NQ + q])


def _sc_p2_body(posbuf, fnbuf, counts, dup_pos, dup_fp,
                ibp, ibf, gb1, gb2, rowb, sem1):
    m, q = _sc_ids()
    base_row = m * (L // CH) + q * (QSZ // CH)
    lane = lax.broadcasted_iota(jnp.int32, (16,), 0)

    for r in range(NQ):
        pltpu.sync_copy(counts.at[m * NQ + r], rowb.at[r])
    P = []
    F = []
    N = []
    for r in range(NQ):
        rv = rowb[r]
        P.append(_lane_extract(rv, 0))
        F.append(_lane_extract(rv, 1))
        N.append(_lane_extract(rv, 2))

    np_tot = P[0] + P[1] + P[2] + P[3]
    nfp_tot = F[0] + F[1] + F[2] + F[3]
    nneg_tot = N[0] + N[1] + N[2] + N[3]
    pc1, pc2, pc3 = P[0], P[0] + P[1], P[0] + P[1] + P[2]
    usefp = nfp_tot > 0
    c1 = jnp.where(usefp, F[0], N[0])
    c2 = c1 + jnp.where(usefp, F[1], N[1])
    c3 = c2 + jnp.where(usefp, F[2], N[2])
    ch_tot = jnp.where(usefp, nfp_tot, nneg_tot)
    ch_extra = jnp.where(usefp, 0, T)
    npc = jnp.maximum(np_tot, 1)
    chc = jnp.maximum(ch_tot, 1)

    def p2_body(c0, carry):
        for k in range(KB):
            for v in range(NV):
                sl = pl.ds(16 * v, 16)
                gi = q * QSZ + c0 * CHB + k * CH + 16 * v + lane
                jp = lax.rem(gi, npc)
                cbp = jnp.where(jp >= pc3, pc3,
                                jnp.where(jp >= pc2, pc2,
                                          jnp.where(jp >= pc1, pc1, 0)))
                one = jnp.int32(1)
                zero = jnp.int32(0)
                qp = (jnp.where(jp >= pc1, one, zero)
                      + jnp.where(jp >= pc2, one, zero)
                      + jnp.where(jp >= pc3, one, zero))
                ibp[k, sl] = m * L + qp * QSZ + (jp - cbp)
                jc = lax.rem(gi, chc)
                cbc = jnp.where(jc >= c3, c3,
                                jnp.where(jc >= c2, c2,
                                          jnp.where(jc >= c1, c1, 0)))
                qc = (jnp.where(jc >= c1, one, zero)
                      + jnp.where(jc >= c2, one, zero)
                      + jnp.where(jc >= c3, one, zero))
                ibf[k, sl] = ch_extra + m * L + qc * QSZ + (jc - cbc)
        hs = []
        for k in range(KB):
            hs.append(pltpu.async_copy(posbuf.at[ibp.at[k]], gb1.at[k], sem1))
            hs.append(pltpu.async_copy(fnbuf.at[ibf.at[k]], gb2.at[k], sem1))
        for h in hs:
            h.wait()
        for k in range(KB):
            for v in range(NV):
                sl = pl.ds(16 * v, 16)
                g1 = gb1[k, sl]
                g2 = gb2[k, sl]
                gb1[k, sl] = jnp.where(np_tot > 0, g1, POS_ALL_NEG)
                gb2[k, sl] = jnp.where(nneg_tot > 0, g2, NEG_ALL_POS)
        row = base_row + c0 * KB
        h1 = pltpu.async_copy(gb1, dup_pos.at[pl.ds(row, KB)], sem1)
        h2 = pltpu.async_copy(gb2, dup_fp.at[pl.ds(row, KB)], sem1)
        h1.wait()
        h2.wait()
        return carry

    lax.fori_loop(0, QSZ // CHB, p2_body, jnp.int32(0))


def _sc_dup(xf2, tf2, af2):
    mesh = plsc.VectorSubcoreMesh(core_axis_name="c", subcore_axis_name="s")
    cparams = pltpu.CompilerParams(needs_layout_passes=False)
    p1 = pl.kernel(
        _sc_p1_body,
        out_type=[
            jax.ShapeDtypeStruct((T,), jnp.float32),
            jax.ShapeDtypeStruct((2 * T,), jnp.float32),
            jax.ShapeDtypeStruct((BS * NQ, 16), jnp.int32),
        ],
        mesh=mesh,
        scratch_types=[
            pltpu.VMEM((KB, CH), jnp.float32),
            pltpu.VMEM((KB, CH), jnp.float32),
            pltpu.VMEM((KB, CH), jnp.float32),
            pltpu.VMEM((KB, CH), jnp.int32),
            pltpu.VMEM((KB, CH), jnp.int32),
            pltpu.VMEM((KB, CH), jnp.int32),
            pltpu.VMEM((16,), jnp.int32),
            pltpu.SemaphoreType.DMA,
        ],
        compiler_params=cparams,
    )
    posbuf, fnbuf, counts = p1(xf2, tf2, af2)
    p2 = pl.kernel(
        _sc_p2_body,
        out_type=[
            jax.ShapeDtypeStruct((ROWS, CH), jnp.float32),
            jax.ShapeDtypeStruct((ROWS, CH), jnp.float32),
        ],
        mesh=mesh,
        scratch_types=[
            pltpu.VMEM((KB, CH), jnp.int32),
            pltpu.VMEM((KB, CH), jnp.int32),
            pltpu.VMEM((KB, CH), jnp.float32),
            pltpu.VMEM((KB, CH), jnp.float32),
            pltpu.VMEM((NQ, 16), jnp.int32),
            pltpu.SemaphoreType.DMA,
        ],
        compiler_params=cparams,
    )
    dup_pos, dup_fp = p2(posbuf, fnbuf, counts)
    return posbuf, fnbuf, dup_pos, dup_fp


def _bce_body(x_ref, t_ref, dp_ref, df_ref, o_ref):
    i = pl.program_id(0)
    x = x_ref[...]
    t = t_ref[...]
    dp = dp_ref[...]
    df = df_ref[...]
    sim = dp * x
    t1 = jnp.maximum(sim, 0.0) - sim * t + jnp.log1p(jnp.exp(-jnp.abs(sim)))
    t2 = jnp.maximum(dp, 0.0) - dp + jnp.log1p(jnp.exp(-jnp.abs(dp)))
    a = dp * df
    t3 = jnp.maximum(a, 0.0) + jnp.log1p(jnp.exp(-jnp.abs(a)))
    blk = x.shape[0]
    p1 = jnp.sum(t1.reshape(blk // 8, 8, 128), axis=0)
    p2 = jnp.sum(t2.reshape(blk // 8, 8, 128), axis=0)
    p3 = jnp.sum(t3.reshape(blk // 8, 8, 128), axis=0)
    acc = jnp.stack([p1, p2, p3])

    @pl.when(i == 0)
    def _():
        o_ref[...] = acc

    @pl.when(i > 0)
    def _():
        o_ref[...] = o_ref[...] + acc


def _bce_sums(x2, t2, dp2, df2):
    blk = 1024
    grid = ROWS // blk
    bs = pl.BlockSpec((blk, CH), lambda i: (i, 0))
    return pl.pallas_call(
        _bce_body,
        grid=(grid,),
        in_specs=[bs, bs, bs, bs],
        out_specs=pl.BlockSpec((3, 8, 128), lambda i: (0, 0, 0)),
        out_shape=jax.ShapeDtypeStruct((3, 8, 128), jnp.float32),
    )(x2, t2, dp2, df2)


def kernel(input, target):
    x = input.astype(jnp.float32)
    t = target.astype(jnp.float32)
    aug = _dilate(t.reshape(BS, H, W))
    x2 = x.reshape(ROWS, CH)
    t2 = t.reshape(ROWS, CH)
    a2 = aug.reshape(ROWS, CH)
    _, _, dp2, df2 = _sc_dup(x2, t2, a2)
    sums = _bce_sums(x2, t2, dp2, df2)
    s = jnp.sum(sums, axis=(1, 2))
    return (s[0] + s[1] + ALPHA_C * s[2]) / jnp.float32(T)
